# baseline (speedup 1.0000x reference)
import jax
import jax.numpy as jnp
from jax.experimental import pallas as pl
from jax.experimental.pallas import tpu as pltpu

_SCALE = 0.5
_EPS = 1e-05

_ROWS = 8192
_COLS = 1024
_CHUNK = 512
_NSTEPS = _ROWS // _CHUNK


def _berhu_body(pred_ref, gt_ref, out_ref, dv_ref, acc_ref):
    i = pl.program_id(0)

    @pl.when(i == 0)
    def _init():
        acc_ref[0] = 0.0
        acc_ref[1] = 0.0
        acc_ref[2] = 0.0

    p = pred_ref[...]
    g = gt_ref[...]
    d = jnp.abs(p - g)
    valid = g > _EPS
    dv = jnp.where(valid, d, 0.0)
    dv_ref[pl.ds(i * _CHUNK, _CHUNK), :] = dv
    acc_ref[0] = jnp.maximum(acc_ref[0], jnp.max(dv))
    acc_ref[1] = acc_ref[1] + jnp.sum(dv)
    acc_ref[2] = acc_ref[2] + jnp.sum(jnp.where(valid, 1.0, 0.0))

    @pl.when(i == _NSTEPS - 1)
    def _finish():
        t = _SCALE * acc_ref[0]
        denom = 2.0 * t + _EPS

        def loop(j, carry):
            w, b = carry
            blk = dv_ref[pl.ds(j * _CHUNK, _CHUNK), :]
            q = jnp.maximum(blk - t, 0.0)
            w = w + jnp.sum(q * q)
            b = b + jnp.sum(jnp.where(blk > t, blk, 0.0))
            return (w, b)

        w, b = jax.lax.fori_loop(0, _NSTEPS, loop, (0.0, 0.0))
        total = acc_ref[1] + (w - _EPS * b) / denom
        out_ref[0] = total / acc_ref[2]


def kernel(pred, gt):
    p2 = pred.reshape(_ROWS, _COLS)
    g2 = gt.reshape(_ROWS, _COLS)
    out = pl.pallas_call(
        _berhu_body,
        grid=(_NSTEPS,),
        in_specs=[
            pl.BlockSpec((_CHUNK, _COLS), lambda i: (i, 0)),
            pl.BlockSpec((_CHUNK, _COLS), lambda i: (i, 0)),
        ],
        out_specs=pl.BlockSpec(memory_space=pltpu.SMEM),
        out_shape=jax.ShapeDtypeStruct((1,), jnp.float32),
        scratch_shapes=[
            pltpu.VMEM((_ROWS, _COLS), jnp.float32),
            pltpu.SMEM((4,), jnp.float32),
        ],
        compiler_params=pltpu.CompilerParams(
            vmem_limit_bytes=56 * 1024 * 1024,
        ),
    )(p2, g2)
    return out[0]
